# baseline (speedup 1.0000x reference)
import math
import functools

import jax
import jax.numpy as jnp
from jax.experimental import pallas as pl


def _router_body(x_ref, wt_ref, tbl_ref, o_ref, *, blk_t, seq_len, n_experts):
    i = pl.program_id(0)
    t_base = jax.lax.rem(i * blk_t, seq_len)

    xb = x_ref[...]
    wt = wt_ref[...]
    tbl = tbl_ref[...]

    hash_codes = jax.lax.dot_general(
        xb, wt, (((1,), (0,)), ((), ())),
        preferred_element_type=jnp.float32)
    s = jnp.sum(hash_codes, axis=1, keepdims=True)
    s = jnp.abs(s)
    s = jnp.mod(s, float(n_experts))
    idx = s.astype(jnp.int32)
    idx = jax.lax.rem(idx, n_experts)

    e_iota = jax.lax.broadcasted_iota(jnp.int32, (blk_t, n_experts), 1)
    onehot = (e_iota == idx).astype(jnp.float32)
    params = jax.lax.dot_general(
        onehot, tbl, (((1,), (0,)), ((), ())),
        preferred_element_type=jnp.float32)
    freqs = params[:, :8]
    amps = params[:, 8:]

    t_iota = jax.lax.broadcasted_iota(jnp.int32, (blk_t, 1), 0).astype(jnp.float32)
    t_norm = (t_iota + jnp.float32(t_base)) * jnp.float32(2.0 * math.pi / seq_len)
    modulation = jnp.sum(amps * jnp.sin(freqs * t_norm), axis=1, keepdims=True)

    o_ref[...] = xb * (1.0 + 0.1 * modulation)


def kernel(x, hash_W, expert_freqs, expert_amplitudes):
    B, T, D = x.shape
    n_experts = expert_freqs.shape[0]
    blk_t = 256

    x2 = x.reshape(B * T, D)
    wt = hash_W.T
    tbl = jnp.concatenate([expert_freqs, expert_amplitudes], axis=1)

    body = functools.partial(_router_body, blk_t=blk_t, seq_len=T,
                             n_experts=n_experts)
    out = pl.pallas_call(
        body,
        grid=((B * T) // blk_t,),
        in_specs=[
            pl.BlockSpec((blk_t, D), lambda i: (i, 0)),
            pl.BlockSpec((D, 16), lambda i: (0, 0)),
            pl.BlockSpec((n_experts, 16), lambda i: (0, 0)),
        ],
        out_specs=pl.BlockSpec((blk_t, D), lambda i: (i, 0)),
        out_shape=jax.ShapeDtypeStruct((B * T, D), jnp.float32),
    )(x2, wt, tbl)
    return out.reshape(B, T, D)


# speedup vs baseline: 1.0917x; 1.0917x over previous
import math
import functools

import jax
import jax.numpy as jnp
from jax.experimental import pallas as pl


def _router_body(x_ref, wt_ref, tblt_ref, o_ref, *, blk_t, seq_len, n_experts):
    i = pl.program_id(0)
    t_base = jax.lax.rem(i * blk_t, seq_len)

    xb = x_ref[...]
    wt = wt_ref[...]
    tblt = tblt_ref[...]

    hash_codes = jax.lax.dot_general(
        xb, wt, (((1,), (0,)), ((), ())),
        preferred_element_type=jnp.float32)
    hc_t = hash_codes.T
    s = jnp.sum(hc_t, axis=0, keepdims=True)
    s = jnp.abs(s)
    s = jnp.mod(s, float(n_experts))
    idx = s.astype(jnp.int32)
    idx = jax.lax.rem(idx, n_experts)

    e_iota = jax.lax.broadcasted_iota(jnp.int32, (n_experts, blk_t), 0)
    onehot = (e_iota == idx).astype(jnp.float32)
    params = jax.lax.dot_general(
        tblt, onehot, (((1,), (0,)), ((), ())),
        preferred_element_type=jnp.float32)
    freqs = params[:8, :]
    amps = params[8:, :]

    t_iota = jax.lax.broadcasted_iota(jnp.int32, (1, blk_t), 1).astype(jnp.float32)
    t_norm = (t_iota + jnp.float32(t_base)) * jnp.float32(2.0 * math.pi / seq_len)
    modulation = jnp.sum(amps * jnp.sin(freqs * t_norm), axis=0, keepdims=True)
    scale = 1.0 + 0.1 * modulation

    o_ref[...] = xb * scale.T


def kernel(x, hash_W, expert_freqs, expert_amplitudes):
    B, T, D = x.shape
    n_experts = expert_freqs.shape[0]
    blk_t = 512

    x2 = x.reshape(B * T, D)
    wt = hash_W.T
    tblt = jnp.concatenate([expert_freqs, expert_amplitudes], axis=1).T

    body = functools.partial(_router_body, blk_t=blk_t, seq_len=T,
                             n_experts=n_experts)
    out = pl.pallas_call(
        body,
        grid=((B * T) // blk_t,),
        in_specs=[
            pl.BlockSpec((blk_t, D), lambda i: (i, 0)),
            pl.BlockSpec((D, 16), lambda i: (0, 0)),
            pl.BlockSpec((16, n_experts), lambda i: (0, 0)),
        ],
        out_specs=pl.BlockSpec((blk_t, D), lambda i: (i, 0)),
        out_shape=jax.ShapeDtypeStruct((B * T, D), jnp.float32),
    )(x2, wt, tblt)
    return out.reshape(B, T, D)
